# baseline (speedup 1.0000x reference)
import functools

import jax
import jax.numpy as jnp
from jax import lax
from jax.experimental import pallas as pl
from jax.experimental.pallas import tpu as pltpu
from jax.experimental.pallas import tpu_sc as plsc

NUM_EXPERTS = 8
TOPK = 2
RBM = 256
BM = 256
BN1 = 4096
BN2 = 2048


def _sc_gather(table, idx):
    V, D = table.shape
    Ptot = idx.shape[0]
    info = plsc.get_sparse_core_info()
    NW = info.num_cores * info.num_subcores
    rows_per_w = Ptot // NW
    CH = 16
    ngrp = rows_per_w // (2 * CH)
    mesh = plsc.VectorSubcoreMesh(core_axis_name="c", subcore_axis_name="s")

    @functools.partial(
        pl.kernel,
        mesh=mesh,
        out_type=jax.ShapeDtypeStruct((Ptot, D), table.dtype),
        scratch_types=[
            pltpu.VMEM((rows_per_w,), jnp.int32),
            pltpu.VMEM((CH, D), table.dtype),
            pltpu.VMEM((CH, D), table.dtype),
            pltpu.SemaphoreType.DMA,
            pltpu.SemaphoreType.DMA,
        ],
    )
    def k(table_hbm, idx_hbm, out_hbm, idx_v, buf0, buf1, sem0, sem1):
        wid = lax.axis_index("s") * info.num_cores + lax.axis_index("c")
        base = wid * rows_per_w
        pltpu.sync_copy(idx_hbm.at[pl.ds(base, rows_per_w)], idx_v)

        def body(i, carry):
            o0 = pl.multiple_of(i * 2 * CH, 8)
            o1 = pl.multiple_of(i * 2 * CH + CH, 8)
            g0 = pltpu.async_copy(table_hbm.at[idx_v.at[pl.ds(o0, CH)]], buf0, sem0)
            g1 = pltpu.async_copy(table_hbm.at[idx_v.at[pl.ds(o1, CH)]], buf1, sem1)
            g0.wait()
            pltpu.sync_copy(buf0, out_hbm.at[pl.ds(base + o0, CH)])
            g1.wait()
            pltpu.sync_copy(buf1, out_hbm.at[pl.ds(base + o1, CH)])
            return carry

        lax.fori_loop(0, ngrp, body, 0)

    return k(table, idx)


def _add_body(a_ref, b_ref, o_ref):
    o_ref[...] = a_ref[...] + b_ref[...]


def _gelu(v):
    return 0.5 * v * (1.0 + lax.erf(v * 0.7071067811865476))


def _router_body(x_ref, wr_ref, br_ref, o_ref):
    logits = jnp.dot(x_ref[...], wr_ref[...], preferred_element_type=jnp.float32)
    logits = logits + br_ref[...]
    ncol = logits.shape[1]
    col = lax.broadcasted_iota(jnp.int32, logits.shape, 1)
    m0 = jnp.max(logits, axis=1, keepdims=True)
    i0 = jnp.min(jnp.where(logits == m0, col, ncol), axis=1, keepdims=True)
    l2 = jnp.where(col == i0, -jnp.float32(jnp.inf), logits)
    m1 = jnp.max(l2, axis=1, keepdims=True)
    i1 = jnp.min(jnp.where(l2 == m1, col, ncol), axis=1, keepdims=True)
    o_ref[...] = jnp.where(col == 0, i0, jnp.where(col == 1, i1, 0)).astype(jnp.int32)


def _ffn1_body(be_ref, xs_ref, w1_ref, b1_ref, h_ref):
    acc = jnp.dot(xs_ref[...].astype(jnp.bfloat16), w1_ref[0],
                  preferred_element_type=jnp.float32)
    h_ref[...] = _gelu(acc + b1_ref[0]).astype(jnp.bfloat16)


def _ffn2_body(be_ref, h_ref, w2_ref, b2_ref, o_ref):
    acc = jnp.dot(h_ref[...], w2_ref[0], preferred_element_type=jnp.float32)
    o_ref[...] = (acc + b2_ref[0]) * 0.5


def _ffn2_body_alias(be_ref, h_ref, w2_ref, b2_ref, odon_ref, o_ref):
    _ffn2_body(be_ref, h_ref, w2_ref, b2_ref, o_ref)


def _mm1(be, xs, W1, b1r):
    P2, EMB = xs.shape
    NE, _, HID = W1.shape
    return pl.pallas_call(
        _ffn1_body,
        grid_spec=pltpu.PrefetchScalarGridSpec(
            num_scalar_prefetch=1,
            grid=(HID // BN1, P2 // BM),
            in_specs=[
                pl.BlockSpec((BM, EMB), lambda n, m, be: (m, 0)),
                pl.BlockSpec((1, EMB, BN1), lambda n, m, be: (be[m], 0, n)),
                pl.BlockSpec((1, 1, BN1), lambda n, m, be: (be[m], 0, n)),
            ],
            out_specs=pl.BlockSpec((BM, BN1), lambda n, m, be: (m, n)),
        ),
        out_shape=jax.ShapeDtypeStruct((P2, HID), jnp.bfloat16),
    )(be, xs, W1, b1r)


def _mm2(be, h, W2, b2r, P, m_off, odon):
    P2, HID = h.shape
    NE, _, EMB = W2.shape
    in_specs = [
        pl.BlockSpec((BM, HID), lambda n, m, be: (m, 0)),
        pl.BlockSpec((1, HID, BN2), lambda n, m, be: (be[m], 0, n)),
        pl.BlockSpec((1, 1, BN2), lambda n, m, be: (be[m], 0, n)),
    ]
    args = [be, h, W2, b2r]
    if odon is None:
        body = _ffn2_body
        io_aliases = {}
    else:
        body = _ffn2_body_alias
        in_specs.append(pl.BlockSpec(memory_space=pl.ANY))
        args.append(odon)
        io_aliases = {4: 0}
    return pl.pallas_call(
        body,
        grid_spec=pltpu.PrefetchScalarGridSpec(
            num_scalar_prefetch=1,
            grid=(EMB // BN2, P2 // BM),
            in_specs=in_specs,
            out_specs=pl.BlockSpec(
                (BM, BN2), lambda n, m, be, off=m_off: (m + off, n)),
        ),
        out_shape=jax.ShapeDtypeStruct((P, EMB), jnp.float32),
        input_output_aliases=io_aliases,
    )(*args)


def kernel(x, Wr, br, W1, b1, W2, b2):
    B, N, EMB = x.shape
    NE, _, HID = W1.shape
    T = B * N
    P = TOPK * T + NE * BM
    num_m = P // BM
    half_m = num_m // 2
    P2 = P // 2
    x_flat = x.reshape(T, EMB)

    wr_pad = jnp.zeros((EMB, 128), Wr.dtype).at[:, :NE].set(Wr)
    br_pad = jnp.full((1, 128), -1e30, br.dtype).at[0, :NE].set(br)
    topk = pl.pallas_call(
        _router_body,
        grid=(T // RBM,),
        in_specs=[
            pl.BlockSpec((RBM, EMB), lambda i: (i, 0)),
            pl.BlockSpec((EMB, 128), lambda i: (0, 0)),
            pl.BlockSpec((1, 128), lambda i: (0, 0)),
        ],
        out_specs=pl.BlockSpec((RBM, 128), lambda i: (i, 0)),
        out_shape=jax.ShapeDtypeStruct((T, 128), jnp.int32),
    )(x_flat, wr_pad, br_pad)
    e0 = topk[:, 0]
    e1 = topk[:, 1]

    ar = jnp.arange(NE, dtype=jnp.int32)
    oh = ((e0[:, None] == ar) | (e1[:, None] == ar)).astype(jnp.int32)
    cum = jnp.cumsum(oh, axis=0)
    counts = cum[-1]
    size_pad = ((counts + BM - 1) // BM) * BM
    start_pad = jnp.concatenate(
        [jnp.zeros((1,), jnp.int32), jnp.cumsum(size_pad)[:-1].astype(jnp.int32)])
    rank = cum - oh
    posm = start_pad[None, :] + rank
    pos0 = jnp.take_along_axis(posm, e0[:, None], axis=1)[:, 0]
    pos1 = jnp.take_along_axis(posm, e1[:, None], axis=1)[:, 0]
    tok = jnp.arange(T, dtype=jnp.int32)
    cat_pos = jnp.concatenate([pos0, pos1])
    cat_tok = jnp.concatenate([tok, tok])
    row_ids = jnp.zeros((P,), jnp.int32).at[cat_pos].set(cat_tok)
    blk_starts = jnp.arange(num_m, dtype=jnp.int32) * BM
    block_expert = jnp.clip(
        (blk_starts[:, None] >= start_pad[None, :]).astype(jnp.int32).sum(axis=1) - 1,
        0, NE - 1)
    be_a = block_expert[:half_m]
    be_b = block_expert[half_m:]

    b1r = b1.reshape(NE, 1, HID)
    b2r = b2.reshape(NE, 1, EMB)
    w1b = W1.astype(jnp.bfloat16)
    w2b = W2.astype(jnp.bfloat16)
    xs_a = _sc_gather(x_flat, row_ids[:P2])
    xs_b = _sc_gather(x_flat, row_ids[P2:])
    h_a = _mm1(be_a, xs_a, w1b, b1r)
    h_b = _mm1(be_b, xs_b, w1b, b1r)
    o_a = _mm2(be_a, h_a, w2b, b2r, P, 0, None)
    o = _mm2(be_b, h_b, w2b, b2r, P, half_m, o_a)

    g = _sc_gather(o, cat_pos)
    half = T // 512
    out = pl.pallas_call(
        _add_body,
        grid=(half,),
        in_specs=[
            pl.BlockSpec((512, EMB), lambda i: (i, 0)),
            pl.BlockSpec((512, EMB), lambda i, h=half: (i + h, 0)),
        ],
        out_specs=pl.BlockSpec((512, EMB), lambda i: (i, 0)),
        out_shape=jax.ShapeDtypeStruct((T, EMB), jnp.float32),
    )(g, g)
    return out.reshape(B, N, EMB)


# speedup vs baseline: 1.1630x; 1.1630x over previous
import functools

import jax
import jax.numpy as jnp
from jax import lax
from jax.experimental import pallas as pl
from jax.experimental.pallas import tpu as pltpu
from jax.experimental.pallas import tpu_sc as plsc

NUM_EXPERTS = 8
TOPK = 2
RBM = 256
BM = 256
BN1 = 2048
BN2 = 1024


def _pack2(a, b):
    ua = lax.bitcast_convert_type(a, jnp.uint32)
    ub = lax.bitcast_convert_type(b, jnp.uint32)
    ra = (ua + 0x7FFF + ((ua >> 16) & 1)) >> 16
    rb = (ub + 0x7FFF + ((ub >> 16) & 1)) >> 16
    return lax.bitcast_convert_type((ra << 16) | rb, jnp.float32)


def _unpack2(w):
    u = lax.bitcast_convert_type(w, jnp.uint32)
    a = lax.bitcast_convert_type((u >> 16) << 16, jnp.float32)
    b = lax.bitcast_convert_type(u << 16, jnp.float32)
    return a, b


def _sc_gather(table, idx):
    V, D = table.shape
    Ptot = idx.shape[0]
    info = plsc.get_sparse_core_info()
    NW = info.num_cores * info.num_subcores
    rows_per_w = Ptot // NW
    CH = 32
    ngrp = rows_per_w // (2 * CH)
    mesh = plsc.VectorSubcoreMesh(core_axis_name="c", subcore_axis_name="s")

    @functools.partial(
        pl.kernel,
        mesh=mesh,
        out_type=jax.ShapeDtypeStruct((Ptot, D), table.dtype),
        scratch_types=[
            pltpu.VMEM((rows_per_w,), jnp.int32),
            pltpu.VMEM((CH, D), table.dtype),
            pltpu.VMEM((CH, D), table.dtype),
            pltpu.SemaphoreType.DMA,
            pltpu.SemaphoreType.DMA,
        ],
    )
    def k(table_hbm, idx_hbm, out_hbm, idx_v, buf0, buf1, sem0, sem1):
        wid = lax.axis_index("s") * info.num_cores + lax.axis_index("c")
        base = wid * rows_per_w
        pltpu.sync_copy(idx_hbm.at[pl.ds(base, rows_per_w)], idx_v)

        def body(i, carry):
            o0 = pl.multiple_of(i * 2 * CH, 8)
            o1 = pl.multiple_of(i * 2 * CH + CH, 8)
            g0 = pltpu.async_copy(table_hbm.at[idx_v.at[pl.ds(o0, CH)]], buf0, sem0)
            g1 = pltpu.async_copy(table_hbm.at[idx_v.at[pl.ds(o1, CH)]], buf1, sem1)
            g0.wait()
            pltpu.sync_copy(buf0, out_hbm.at[pl.ds(base + o0, CH)])
            g1.wait()
            pltpu.sync_copy(buf1, out_hbm.at[pl.ds(base + o1, CH)])
            return carry

        lax.fori_loop(0, ngrp, body, 0)

    return k(table, idx)


def _gelu(v):
    return 0.5 * v * (1.0 + lax.erf(v * 0.7071067811865476))


def _router_body(x_ref, wr_ref, br_ref, o_ref, xp_ref):
    logits = jnp.dot(x_ref[...], wr_ref[...], preferred_element_type=jnp.float32)
    logits = logits + br_ref[...]
    ncol = logits.shape[1]
    col = lax.broadcasted_iota(jnp.int32, logits.shape, 1)
    m0 = jnp.max(logits, axis=1, keepdims=True)
    i0 = jnp.min(jnp.where(logits == m0, col, ncol), axis=1, keepdims=True)
    l2 = jnp.where(col == i0, -jnp.float32(jnp.inf), logits)
    m1 = jnp.max(l2, axis=1, keepdims=True)
    i1 = jnp.min(jnp.where(l2 == m1, col, ncol), axis=1, keepdims=True)
    o_ref[...] = jnp.where(col == 0, i0, jnp.where(col == 1, i1, 0)).astype(jnp.int32)
    half = x_ref.shape[1] // 2
    xp_ref[...] = _pack2(x_ref[:, :half], x_ref[:, half:])


def _ffn1_body(be_ref, xp_ref, w1_ref, b1_ref, h_ref, wbf_ref):
    m = pl.program_id(1)
    prev = be_ref[jnp.maximum(m - 1, 0)]

    @pl.when((m == 0) | (be_ref[m] != prev))
    def _cast():
        wbf_ref[...] = w1_ref[0].astype(jnp.bfloat16)

    half = xp_ref.shape[1]
    a, b = _unpack2(xp_ref[...])
    acc = jnp.dot(a.astype(jnp.bfloat16), wbf_ref[:half],
                  preferred_element_type=jnp.float32)
    acc += jnp.dot(b.astype(jnp.bfloat16), wbf_ref[half:],
                   preferred_element_type=jnp.float32)
    h_ref[...] = _gelu(acc + b1_ref[0]).astype(jnp.bfloat16)


def _ffn2_body(be_ref, h_ref, w2_ref, b2_ref, o_ref, wbf_ref):
    m = pl.program_id(1)
    prev = be_ref[jnp.maximum(m - 1, 0)]

    @pl.when((m == 0) | (be_ref[m] != prev))
    def _cast():
        wbf_ref[...] = w2_ref[0].astype(jnp.bfloat16)

    acc = jnp.dot(h_ref[...], wbf_ref[...], preferred_element_type=jnp.float32)
    acc = (acc + b2_ref[0]) * 0.5
    half = acc.shape[1] // 2
    o_ref[...] = _pack2(acc[:, :half], acc[:, half:])


def _add_body(g0_ref, g1_ref, o_ref):
    D2 = g0_ref.shape[1]
    nb = D2 // 2
    for n in range(2):
        w0 = g0_ref[:, n * nb:(n + 1) * nb]
        w1 = g1_ref[:, n * nb:(n + 1) * nb]
        a0, b0 = _unpack2(w0)
        a1, b1 = _unpack2(w1)
        o_ref[:, 2 * n * nb:(2 * n + 1) * nb] = a0 + a1
        o_ref[:, (2 * n + 1) * nb:(2 * n + 2) * nb] = b0 + b1


def kernel(x, Wr, br, W1, b1, W2, b2):
    B, N, EMB = x.shape
    NE, _, HID = W1.shape
    T = B * N
    P = TOPK * T + NE * BM
    num_m = P // BM
    x_flat = x.reshape(T, EMB)

    wr_pad = jnp.zeros((EMB, 128), Wr.dtype).at[:, :NE].set(Wr)
    br_pad = jnp.full((1, 128), -1e30, br.dtype).at[0, :NE].set(br)
    topk, xpk = pl.pallas_call(
        _router_body,
        grid=(T // RBM,),
        in_specs=[
            pl.BlockSpec((RBM, EMB), lambda i: (i, 0)),
            pl.BlockSpec((EMB, 128), lambda i: (0, 0)),
            pl.BlockSpec((1, 128), lambda i: (0, 0)),
        ],
        out_specs=(
            pl.BlockSpec((RBM, 128), lambda i: (i, 0)),
            pl.BlockSpec((RBM, EMB // 2), lambda i: (i, 0)),
        ),
        out_shape=(
            jax.ShapeDtypeStruct((T, 128), jnp.int32),
            jax.ShapeDtypeStruct((T, EMB // 2), jnp.float32),
        ),
    )(x_flat, wr_pad, br_pad)
    e0 = topk[:, 0]
    e1 = topk[:, 1]

    ar = jnp.arange(NE, dtype=jnp.int32)
    oh = ((e0[:, None] == ar) | (e1[:, None] == ar)).astype(jnp.int32)
    cum = jnp.cumsum(oh, axis=0)
    counts = cum[-1]
    size_pad = ((counts + BM - 1) // BM) * BM
    start_pad = jnp.concatenate(
        [jnp.zeros((1,), jnp.int32), jnp.cumsum(size_pad)[:-1].astype(jnp.int32)])
    rank = cum - oh
    posm = start_pad[None, :] + rank
    pos0 = jnp.take_along_axis(posm, e0[:, None], axis=1)[:, 0]
    pos1 = jnp.take_along_axis(posm, e1[:, None], axis=1)[:, 0]
    tok = jnp.arange(T, dtype=jnp.int32)
    cat_pos = jnp.concatenate([pos0, pos1])
    cat_tok = jnp.concatenate([tok, tok])
    row_ids = jnp.zeros((P,), jnp.int32).at[cat_pos].set(cat_tok)
    blk_starts = jnp.arange(num_m, dtype=jnp.int32) * BM
    block_expert = jnp.clip(
        (blk_starts[:, None] >= start_pad[None, :]).astype(jnp.int32).sum(axis=1) - 1,
        0, NE - 1)

    xs = _sc_gather(xpk, row_ids)

    h = pl.pallas_call(
        _ffn1_body,
        grid_spec=pltpu.PrefetchScalarGridSpec(
            num_scalar_prefetch=1,
            grid=(HID // BN1, num_m),
            in_specs=[
                pl.BlockSpec((BM, EMB // 2), lambda n, m, be: (m, 0)),
                pl.BlockSpec((1, EMB, BN1), lambda n, m, be: (be[m], 0, n)),
                pl.BlockSpec((1, 1, BN1), lambda n, m, be: (be[m], 0, n)),
            ],
            out_specs=pl.BlockSpec((BM, BN1), lambda n, m, be: (m, n)),
            scratch_shapes=[pltpu.VMEM((EMB, BN1), jnp.bfloat16)],
        ),
        out_shape=jax.ShapeDtypeStruct((P, HID), jnp.bfloat16),
    )(block_expert, xs, W1, b1.reshape(NE, 1, HID))

    o = pl.pallas_call(
        _ffn2_body,
        grid_spec=pltpu.PrefetchScalarGridSpec(
            num_scalar_prefetch=1,
            grid=(EMB // BN2, num_m),
            in_specs=[
                pl.BlockSpec((BM, HID), lambda n, m, be: (m, 0)),
                pl.BlockSpec((1, HID, BN2), lambda n, m, be: (be[m], 0, n)),
                pl.BlockSpec((1, 1, BN2), lambda n, m, be: (be[m], 0, n)),
            ],
            out_specs=pl.BlockSpec((BM, BN2 // 2), lambda n, m, be: (m, n)),
            scratch_shapes=[pltpu.VMEM((HID, BN2), jnp.bfloat16)],
        ),
        out_shape=jax.ShapeDtypeStruct((P, EMB // 2), jnp.float32),
    )(block_expert, h, W2, b2.reshape(NE, 1, EMB))

    g = _sc_gather(o, cat_pos)
    half = T // 512
    out = pl.pallas_call(
        _add_body,
        grid=(half,),
        in_specs=[
            pl.BlockSpec((512, EMB // 2), lambda i: (i, 0)),
            pl.BlockSpec((512, EMB // 2), lambda i, h=half: (i + h, 0)),
        ],
        out_specs=pl.BlockSpec((512, EMB), lambda i: (i, 0)),
        out_shape=jax.ShapeDtypeStruct((T, EMB), jnp.float32),
    )(g, g)
    return out.reshape(B, N, EMB)
